# baseline (speedup 1.0000x reference)
import functools

import jax
import jax.numpy as jnp
from jax import lax
from jax.experimental import pallas as pl
from jax.experimental.pallas import tpu as pltpu
from jax.experimental.pallas import tpu_sc as plsc

BATCH = 4
SEQ = 8192
DIM = 1024

_info = plsc.get_sparse_core_info()
NC, NS = _info.num_cores, _info.num_subcores
NW = NC * NS
ROWS_PER_W = SEQ // NW
SH_ROWS = 48
TS_ROWS = ROWS_PER_W - SH_ROWS
TS_CHUNKS = (72, 72, 64)

_mesh = plsc.VectorSubcoreMesh(core_axis_name="c", subcore_axis_name="s")


@functools.partial(
    pl.kernel,
    mesh=_mesh,
    out_type=jax.ShapeDtypeStruct((BATCH, SEQ, DIM), jnp.float32),
    scratch_types=[
        pltpu.VMEM((TS_CHUNKS[0], DIM), jnp.float32),
        pltpu.VMEM_SHARED((NS, SH_ROWS, DIM), jnp.float32),
        pltpu.SemaphoreType.DMA,
        pltpu.SemaphoreType.DMA,
    ],
)
def _broadcast_rows(table_hbm, out_hbm, buf, shared, sem_g, sem_w):
    cid = lax.axis_index("c")
    sid = lax.axis_index("s")
    wid = sid * NC + cid
    base = wid * ROWS_PER_W

    sh = shared.at[sid]
    gh = pltpu.async_copy(table_hbm.at[pl.ds(base + TS_ROWS, SH_ROWS)], sh, sem_g)

    wh = []
    off = 0
    for ci, sz in enumerate(TS_CHUNKS):
        r = base + off
        dst_buf = buf if sz == TS_CHUNKS[0] else buf.at[pl.ds(0, sz)]
        pltpu.sync_copy(table_hbm.at[pl.ds(r, sz)], dst_buf)
        for b in range(BATCH):
            pltpu.sync_copy(dst_buf, out_hbm.at[b, pl.ds(r, sz)])
        if ci == 0:
            gh.wait()
            wh = [
                pltpu.async_copy(
                    sh, out_hbm.at[b, pl.ds(base + TS_ROWS, SH_ROWS)], sem_w
                )
                for b in range(BATCH)
            ]
        off += sz

    for h in wh:
        h.wait()


def kernel(x, pos_embedding):
    del x
    return _broadcast_rows(pos_embedding)
